# baseline (speedup 1.0000x reference)
import jax
import jax.numpy as jnp
from jax import lax
from jax.experimental import pallas as pl
from jax.experimental.pallas import tpu as pltpu
from jax.experimental.pallas import tpu_sc as plsc

NC, NS, L = 2, 16, 16
NW = NC * NS
B = 16384
K = 50
D = 16
RPW = B // NW
SUB = 128
NSUB = RPW // SUB
NG = SUB // L
REGS_COEF = 1e-05


def _sc_body(user_hbm, pos_hbm, negs_hbm, uemb_hbm, iemb_hbm,
             diff_hbm, sq_hbm,
             u_idx_v, p_idx_v, n_idx_v, u_rows, p_rows, n_rows,
             bid_v, diff_v, nbest, sq_stage, sem):
    wid = lax.axis_index("s") * NC + lax.axis_index("c")
    iota = lax.iota(jnp.int32, L)
    dcol = [jnp.full((L,), d, jnp.int32) for d in range(D)]

    def sub_body(s, sq_total):
        base = wid * RPW + s * SUB
        pltpu.sync_copy(user_hbm.at[pl.ds(base, SUB)], u_idx_v)
        pltpu.sync_copy(pos_hbm.at[pl.ds(base, SUB)], p_idx_v)
        pltpu.sync_copy(negs_hbm.at[pl.ds(base * K, SUB * K)], n_idx_v)
        cu = pltpu.async_copy(uemb_hbm.at[u_idx_v], u_rows, sem)
        cp = pltpu.async_copy(iemb_hbm.at[p_idx_v], p_rows, sem)
        cn = pltpu.async_copy(iemb_hbm.at[n_idx_v], n_rows, sem)
        cu.wait()
        cp.wait()
        cn.wait()

        def g_body(g, sq_acc):
            rows = g * L + iota
            rows_k = rows * K
            ud = [plsc.load_gather(u_rows, [rows, dcol[d]]) for d in range(D)]
            ps = jnp.zeros((L,), jnp.float32)
            for d in range(D):
                pv = plsc.load_gather(p_rows, [rows, dcol[d]])
                ps = ps + ud[d] * pv
                sq_acc = sq_acc + ud[d] * ud[d] + pv * pv

            def j_body(j, c):
                best, bid = c
                frow = rows_k + j
                acc = jnp.zeros((L,), jnp.float32)
                for d in range(D):
                    nv = plsc.load_gather(n_rows, [frow, dcol[d]])
                    acc = acc + ud[d] * nv
                ids = plsc.load_gather(n_idx_v, [frow])
                m = acc > best
                return jnp.where(m, acc, best), jnp.where(m, ids, bid)

            best, bid = lax.fori_loop(
                0, K, j_body,
                (jnp.full((L,), -1e30, jnp.float32),
                 jnp.zeros((L,), jnp.int32)))
            bid_v[pl.ds(g * L, L)] = bid
            diff_v[pl.ds(g * L, L)] = ps - best
            return sq_acc

        sq_total = lax.fori_loop(0, NG, g_body, sq_total)

        cb = pltpu.async_copy(iemb_hbm.at[bid_v], nbest, sem)
        cb.wait()

        def r_body(r, sq_acc):
            v = nbest[r, :]
            return sq_acc + v * v

        sq_total = lax.fori_loop(0, SUB, r_body, sq_total)

        pltpu.sync_copy(diff_v, diff_hbm.at[pl.ds(base, SUB)])
        return sq_total

    sq_total = lax.fori_loop(0, NSUB, sub_body, jnp.zeros((L,), jnp.float32))
    sq_stage[...] = sq_total
    pltpu.sync_copy(sq_stage, sq_hbm.at[wid])


_sc_call = pl.kernel(
    _sc_body,
    out_type=(jax.ShapeDtypeStruct((B,), jnp.float32),
              jax.ShapeDtypeStruct((NW, L), jnp.float32)),
    mesh=plsc.VectorSubcoreMesh(core_axis_name="c", subcore_axis_name="s",
                                num_cores=NC, num_subcores=NS),
    scratch_types=[
        pltpu.VMEM((SUB,), jnp.int32),
        pltpu.VMEM((SUB,), jnp.int32),
        pltpu.VMEM((SUB * K,), jnp.int32),
        pltpu.VMEM((SUB, D), jnp.float32),
        pltpu.VMEM((SUB, D), jnp.float32),
        pltpu.VMEM((SUB * K, D), jnp.float32),
        pltpu.VMEM((SUB,), jnp.int32),
        pltpu.VMEM((SUB,), jnp.float32),
        pltpu.VMEM((SUB, D), jnp.float32),
        pltpu.VMEM((L,), jnp.float32),
        pltpu.SemaphoreType.DMA,
    ],
    compiler_params=pltpu.CompilerParams(needs_layout_passes=False,
                                         use_tc_tiling_on_sc=False),
)


def _tc_body(diff_ref, sq_ref, loss_ref, reg_ref):
    z = -diff_ref[...]
    sp = jnp.maximum(z, 0.0) + jnp.log1p(jnp.exp(-jnp.abs(z)))
    loss_ref[...] = jnp.sum(sp).reshape(1, 1) / B
    reg_ref[...] = (REGS_COEF * 0.5 / B) * jnp.sum(sq_ref[...]).reshape(1, 1)


@jax.jit
def _run(user, pos, negs, user_embedding, item_embedding):
    user = user.astype(jnp.int32)
    pos = pos.astype(jnp.int32)
    negs = negs.astype(jnp.int32).reshape(B * K)
    diff, sq = _sc_call(user, pos, negs, user_embedding, item_embedding)
    loss, reg = pl.pallas_call(
        _tc_body,
        out_shape=(jax.ShapeDtypeStruct((1, 1), jnp.float32),
                   jax.ShapeDtypeStruct((1, 1), jnp.float32)),
    )(diff.reshape(128, 128), sq)
    return loss[0, 0], reg[0, 0]


def kernel(user, pos, negs, user_embedding, item_embedding):
    return _run(user, pos, negs, user_embedding, item_embedding)


# speedup vs baseline: 1.1301x; 1.1301x over previous
import jax
import jax.numpy as jnp
from jax import lax
from jax.experimental import pallas as pl
from jax.experimental.pallas import tpu as pltpu
from jax.experimental.pallas import tpu_sc as plsc

NC, NS, L = 2, 16, 16
NW = NC * NS
B = 16384
K = 50
D = 16
RPW = B // NW
SUB = 128
NSUB = RPW // SUB
NG = SUB // L
REGS_COEF = 1e-05


def _sc_body(user_hbm, pos_hbm, negs_hbm, uemb_hbm, iemb_hbm,
             diff_hbm, sq_hbm,
             u_idx_v, p_idx_v, n_idx_v, u_rows, p_rows, n_rows,
             bid_v, diff_v, nbest, sq_stage, sem):
    wid = lax.axis_index("s") * NC + lax.axis_index("c")
    iota = lax.iota(jnp.int32, L)
    dcol = [jnp.full((L,), d, jnp.int32) for d in range(D)]

    def sub_body(s, sq_total):
        base = wid * RPW + s * SUB
        pltpu.sync_copy(user_hbm.at[pl.ds(base, SUB)], u_idx_v)
        pltpu.sync_copy(pos_hbm.at[pl.ds(base, SUB)], p_idx_v)
        pltpu.sync_copy(negs_hbm.at[:, pl.ds(base, SUB)], n_idx_v)
        cu = pltpu.async_copy(uemb_hbm.at[u_idx_v], u_rows, sem)
        cp = pltpu.async_copy(iemb_hbm.at[p_idx_v], p_rows, sem)

        def issue_body(j, _):
            pltpu.async_copy(iemb_hbm.at[n_idx_v.at[j]],
                             n_rows.at[pl.ds(j * SUB, SUB), :], sem)
            return 0

        lax.fori_loop(0, K, issue_body, 0)
        cu.wait()
        cp.wait()

        def drain_body(j, _):
            pltpu.make_async_copy(iemb_hbm.at[n_idx_v.at[0]],
                                  n_rows.at[pl.ds(0, SUB), :], sem).wait()
            return 0

        lax.fori_loop(0, K, drain_body, 0)

        def g_body(g, sq_acc):
            rows = g * L + iota
            ud = [plsc.load_gather(u_rows, [rows, dcol[d]]) for d in range(D)]
            ps = jnp.zeros((L,), jnp.float32)
            for d in range(D):
                pv = plsc.load_gather(p_rows, [rows, dcol[d]])
                ps = ps + ud[d] * pv
                sq_acc = sq_acc + ud[d] * ud[d] + pv * pv

            def j_body(j, c):
                best, bid = c
                frow = j * SUB + rows
                jsplat = jnp.full((L,), j, jnp.int32)
                acc = jnp.zeros((L,), jnp.float32)
                for d in range(D):
                    nv = plsc.load_gather(n_rows, [frow, dcol[d]])
                    acc = acc + ud[d] * nv
                ids = plsc.load_gather(n_idx_v, [jsplat, rows])
                m = acc > best
                return jnp.where(m, acc, best), jnp.where(m, ids, bid)

            best, bid = lax.fori_loop(
                0, K, j_body,
                (jnp.full((L,), -1e30, jnp.float32),
                 jnp.zeros((L,), jnp.int32)))
            bid_v[pl.ds(g * L, L)] = bid
            diff_v[pl.ds(g * L, L)] = ps - best
            return sq_acc

        sq_total = lax.fori_loop(0, NG, g_body, sq_total)

        cb = pltpu.async_copy(iemb_hbm.at[bid_v], nbest, sem)
        cb.wait()

        def r_body(r, sq_acc):
            v = nbest[r, :]
            return sq_acc + v * v

        sq_total = lax.fori_loop(0, SUB, r_body, sq_total)

        pltpu.sync_copy(diff_v, diff_hbm.at[pl.ds(base, SUB)])
        return sq_total

    sq_total = lax.fori_loop(0, NSUB, sub_body, jnp.zeros((L,), jnp.float32))
    sq_stage[...] = sq_total
    pltpu.sync_copy(sq_stage, sq_hbm.at[wid])


_sc_call = pl.kernel(
    _sc_body,
    out_type=(jax.ShapeDtypeStruct((B,), jnp.float32),
              jax.ShapeDtypeStruct((NW, L), jnp.float32)),
    mesh=plsc.VectorSubcoreMesh(core_axis_name="c", subcore_axis_name="s",
                                num_cores=NC, num_subcores=NS),
    scratch_types=[
        pltpu.VMEM((SUB,), jnp.int32),
        pltpu.VMEM((SUB,), jnp.int32),
        pltpu.VMEM((K, SUB), jnp.int32),
        pltpu.VMEM((SUB, D), jnp.float32),
        pltpu.VMEM((SUB, D), jnp.float32),
        pltpu.VMEM((SUB * K, D), jnp.float32),
        pltpu.VMEM((SUB,), jnp.int32),
        pltpu.VMEM((SUB,), jnp.float32),
        pltpu.VMEM((SUB, D), jnp.float32),
        pltpu.VMEM((L,), jnp.float32),
        pltpu.SemaphoreType.DMA,
    ],
    compiler_params=pltpu.CompilerParams(needs_layout_passes=False,
                                         use_tc_tiling_on_sc=False),
)


def _tc_body(diff_ref, sq_ref, loss_ref, reg_ref):
    z = -diff_ref[...]
    sp = jnp.maximum(z, 0.0) + jnp.log1p(jnp.exp(-jnp.abs(z)))
    loss_ref[...] = jnp.sum(sp).reshape(1, 1) / B
    reg_ref[...] = (REGS_COEF * 0.5 / B) * jnp.sum(sq_ref[...]).reshape(1, 1)


@jax.jit
def _run(user, pos, negs, user_embedding, item_embedding):
    user = user.astype(jnp.int32)
    pos = pos.astype(jnp.int32)
    negs_t = negs.astype(jnp.int32).T
    diff, sq = _sc_call(user, pos, negs_t, user_embedding, item_embedding)
    loss, reg = pl.pallas_call(
        _tc_body,
        out_shape=(jax.ShapeDtypeStruct((1, 1), jnp.float32),
                   jax.ShapeDtypeStruct((1, 1), jnp.float32)),
    )(diff.reshape(128, 128), sq)
    return loss[0, 0], reg[0, 0]


def kernel(user, pos, negs, user_embedding, item_embedding):
    return _run(user, pos, negs, user_embedding, item_embedding)


# speedup vs baseline: 1.1362x; 1.0053x over previous
import jax
import jax.numpy as jnp
from jax import lax
from jax.experimental import pallas as pl
from jax.experimental.pallas import tpu as pltpu
from jax.experimental.pallas import tpu_sc as plsc

NC, NS, L = 2, 16, 16
NW = NC * NS
B = 16384
K = 50
D = 16
RPW = B // NW
SUB = 128
NSUB = RPW // SUB
NG = SUB // L
REGS_COEF = 1e-05


def _sc_body(user_hbm, pos_hbm, negs_hbm, uemb_hbm, iemb_hbm,
             diff_hbm, sq_hbm,
             u_idx_v, p_idx_v, n_idx_v, u_rows, p_rows, n_rows,
             bid_v, diff_v, nbest, sq_stage, sem, isem):
    wid = lax.axis_index("s") * NC + lax.axis_index("c")
    iota = lax.iota(jnp.int32, L)
    dcol = [jnp.full((L,), d, jnp.int32) for d in range(D)]

    def sub_body(s, sq_total):
        base = wid * RPW + s * SUB
        pltpu.sync_copy(user_hbm.at[pl.ds(base, SUB)], u_idx_v)
        pltpu.sync_copy(pos_hbm.at[pl.ds(base, SUB)], p_idx_v)

        def idx_body(j, _):
            pltpu.async_copy(negs_hbm.at[pl.ds(j * B + base, SUB)],
                             n_idx_v.at[j], isem)
            return 0

        lax.fori_loop(0, K, idx_body, 0)
        cu = pltpu.async_copy(uemb_hbm.at[u_idx_v], u_rows, sem)
        cp = pltpu.async_copy(iemb_hbm.at[p_idx_v], p_rows, sem)

        def idx_drain(j, _):
            pltpu.make_async_copy(negs_hbm.at[pl.ds(0, SUB)],
                                  n_idx_v.at[0], isem).wait()
            return 0

        lax.fori_loop(0, K, idx_drain, 0)

        def issue_body(j, _):
            pltpu.async_copy(iemb_hbm.at[n_idx_v.at[j]],
                             n_rows.at[pl.ds(j * SUB, SUB), :], sem)
            return 0

        lax.fori_loop(0, K, issue_body, 0)
        cu.wait()
        cp.wait()

        def drain_body(j, _):
            pltpu.make_async_copy(iemb_hbm.at[n_idx_v.at[0]],
                                  n_rows.at[pl.ds(0, SUB), :], sem).wait()
            return 0

        lax.fori_loop(0, K, drain_body, 0)

        def g_body(g, sq_acc):
            rows = g * L + iota
            ud = [plsc.load_gather(u_rows, [rows, dcol[d]]) for d in range(D)]
            ps = jnp.zeros((L,), jnp.float32)
            for d in range(D):
                pv = plsc.load_gather(p_rows, [rows, dcol[d]])
                ps = ps + ud[d] * pv
                sq_acc = sq_acc + ud[d] * ud[d] + pv * pv

            def j_body(j, c):
                best, bid = c
                frow = j * SUB + rows
                jsplat = jnp.full((L,), j, jnp.int32)
                acc = jnp.zeros((L,), jnp.float32)
                for d in range(D):
                    nv = plsc.load_gather(n_rows, [frow, dcol[d]])
                    acc = acc + ud[d] * nv
                ids = plsc.load_gather(n_idx_v, [jsplat, rows])
                m = acc > best
                return jnp.where(m, acc, best), jnp.where(m, ids, bid)

            best, bid = lax.fori_loop(
                0, K, j_body,
                (jnp.full((L,), -1e30, jnp.float32),
                 jnp.zeros((L,), jnp.int32)))
            bid_v[pl.ds(g * L, L)] = bid
            diff_v[pl.ds(g * L, L)] = ps - best
            return sq_acc

        sq_total = lax.fori_loop(0, NG, g_body, sq_total)

        cb = pltpu.async_copy(iemb_hbm.at[bid_v], nbest, sem)
        cb.wait()

        def r_body(r, sq_acc):
            v = nbest[r, :]
            return sq_acc + v * v

        sq_total = lax.fori_loop(0, SUB, r_body, sq_total)

        pltpu.sync_copy(diff_v, diff_hbm.at[pl.ds(base, SUB)])
        return sq_total

    sq_total = lax.fori_loop(0, NSUB, sub_body, jnp.zeros((L,), jnp.float32))
    sq_stage[...] = sq_total
    pltpu.sync_copy(sq_stage, sq_hbm.at[wid])


_sc_call = pl.kernel(
    _sc_body,
    out_type=(jax.ShapeDtypeStruct((B,), jnp.float32),
              jax.ShapeDtypeStruct((NW, L), jnp.float32)),
    mesh=plsc.VectorSubcoreMesh(core_axis_name="c", subcore_axis_name="s",
                                num_cores=NC, num_subcores=NS),
    scratch_types=[
        pltpu.VMEM((SUB,), jnp.int32),
        pltpu.VMEM((SUB,), jnp.int32),
        pltpu.VMEM((K, SUB), jnp.int32),
        pltpu.VMEM((SUB, D), jnp.float32),
        pltpu.VMEM((SUB, D), jnp.float32),
        pltpu.VMEM((SUB * K, D), jnp.float32),
        pltpu.VMEM((SUB,), jnp.int32),
        pltpu.VMEM((SUB,), jnp.float32),
        pltpu.VMEM((SUB, D), jnp.float32),
        pltpu.VMEM((L,), jnp.float32),
        pltpu.SemaphoreType.DMA,
        pltpu.SemaphoreType.DMA,
    ],
    compiler_params=pltpu.CompilerParams(needs_layout_passes=False,
                                         use_tc_tiling_on_sc=False),
)


def _flat_body(n_ref, o_ref):
    o_ref[...] = n_ref[...].reshape(K * B)


_flatten_negs = pl.pallas_call(
    _flat_body,
    out_shape=jax.ShapeDtypeStruct((K * B,), jnp.int32),
)


def _tc_body(diff_ref, sq_ref, loss_ref, reg_ref):
    z = -diff_ref[...]
    sp = jnp.maximum(z, 0.0) + jnp.log1p(jnp.exp(-jnp.abs(z)))
    loss_ref[...] = jnp.sum(sp).reshape(1, 1) / B
    reg_ref[...] = (REGS_COEF * 0.5 / B) * jnp.sum(sq_ref[...]).reshape(1, 1)


@jax.jit
def _run(user, pos, negs, user_embedding, item_embedding):
    user = user.astype(jnp.int32)
    pos = pos.astype(jnp.int32)
    negs_flat = _flatten_negs(negs.astype(jnp.int32).T)
    diff, sq = _sc_call(user, pos, negs_flat, user_embedding, item_embedding)
    loss, reg = pl.pallas_call(
        _tc_body,
        out_shape=(jax.ShapeDtypeStruct((1, 1), jnp.float32),
                   jax.ShapeDtypeStruct((1, 1), jnp.float32)),
    )(diff.reshape(128, 128), sq)
    return loss[0, 0], reg[0, 0]


def kernel(user, pos, negs, user_embedding, item_embedding):
    return _run(user, pos, negs, user_embedding, item_embedding)


# speedup vs baseline: 2.1342x; 1.8784x over previous
import jax
import jax.numpy as jnp
from jax import lax
from jax.experimental import pallas as pl
from jax.experimental.pallas import tpu as pltpu
from jax.experimental.pallas import tpu_sc as plsc

NC, NS, L = 2, 16, 16
NW = NC * NS
B = 16384
K = 50
D = 16
RPW = B // NW
SUB = 128
NSUB = RPW // SUB
NG = SUB // L
REGS_COEF = 1e-05


def _sc_body(user_hbm, pos_hbm, negs_hbm, uemb_hbm, iemb_hbm,
             diff_hbm, sq_hbm,
             u_idx_v, p_idx_v, n_idx_v, u_rows, p_rows, n_rows,
             bid_v, diff_v, nbest, sq_stage, sem, isem):
    wid = lax.axis_index("s") * NC + lax.axis_index("c")
    iota = lax.iota(jnp.int32, L)
    dcol = [jnp.full((L,), d, jnp.int32) for d in range(D)]

    def sub_body(s, sq_total):
        base = wid * RPW + s * SUB
        pltpu.sync_copy(user_hbm.at[pl.ds(base, SUB)], u_idx_v)
        pltpu.sync_copy(pos_hbm.at[pl.ds(base, SUB)], p_idx_v)

        def idx_body(j, _):
            pltpu.async_copy(negs_hbm.at[pl.ds(j * B + base, SUB)],
                             n_idx_v.at[j], isem)
            return 0

        lax.fori_loop(0, K, idx_body, 0)
        cu = pltpu.async_copy(uemb_hbm.at[u_idx_v], u_rows, sem)
        cp = pltpu.async_copy(iemb_hbm.at[p_idx_v], p_rows, sem)

        def idx_drain(j, _):
            pltpu.make_async_copy(negs_hbm.at[pl.ds(0, SUB)],
                                  n_idx_v.at[0], isem).wait()
            return 0

        lax.fori_loop(0, K, idx_drain, 0)

        def issue_body(j, _):
            pltpu.async_copy(iemb_hbm.at[n_idx_v.at[j]],
                             n_rows.at[pl.ds(j * SUB, SUB), :], sem)
            return 0

        lax.fori_loop(0, K, issue_body, 0)
        cu.wait()
        cp.wait()

        def drain_body(j, _):
            pltpu.make_async_copy(iemb_hbm.at[n_idx_v.at[0]],
                                  n_rows.at[pl.ds(0, SUB), :], sem).wait()
            return 0

        lax.fori_loop(0, K, drain_body, 0)

        def g_body(g, sq_acc):
            rows = g * L + iota
            ud = [plsc.load_gather(u_rows, [rows, dcol[d]]) for d in range(D)]
            ps = jnp.zeros((L,), jnp.float32)
            for d in range(D):
                pv = plsc.load_gather(p_rows, [rows, dcol[d]])
                ps = ps + ud[d] * pv
                sq_acc = sq_acc + ud[d] * ud[d] + pv * pv

            def j_body(j, c):
                best, bid = c
                frow = j * SUB + rows
                jsplat = jnp.full((L,), j, jnp.int32)
                acc = jnp.zeros((L,), jnp.float32)
                for d in range(D):
                    nv = plsc.load_gather(n_rows, [frow, dcol[d]])
                    acc = acc + ud[d] * nv
                ids = plsc.load_gather(n_idx_v, [jsplat, rows])
                m = acc > best
                return jnp.where(m, acc, best), jnp.where(m, ids, bid)

            best, bid = lax.fori_loop(
                0, K, j_body,
                (jnp.full((L,), -1e30, jnp.float32),
                 jnp.zeros((L,), jnp.int32)))
            bid_v[pl.ds(g * L, L)] = bid
            diff_v[pl.ds(g * L, L)] = ps - best
            return sq_acc

        sq_total = lax.fori_loop(0, NG, g_body, sq_total)

        cb = pltpu.async_copy(iemb_hbm.at[bid_v], nbest, sem)
        cb.wait()

        def r_body(r, sq_acc):
            v = nbest[r, :]
            return sq_acc + v * v

        sq_total = lax.fori_loop(0, SUB, r_body, sq_total)

        pltpu.sync_copy(diff_v, diff_hbm.at[pl.ds(base, SUB)])
        return sq_total

    sq_total = lax.fori_loop(0, NSUB, sub_body, jnp.zeros((L,), jnp.float32))
    sq_stage[...] = sq_total
    pltpu.sync_copy(sq_stage, sq_hbm.at[wid])


_sc_call = pl.kernel(
    _sc_body,
    out_type=(jax.ShapeDtypeStruct((B,), jnp.float32),
              jax.ShapeDtypeStruct((NW, L), jnp.float32)),
    mesh=plsc.VectorSubcoreMesh(core_axis_name="c", subcore_axis_name="s",
                                num_cores=NC, num_subcores=NS),
    scratch_types=[
        pltpu.VMEM((SUB,), jnp.int32),
        pltpu.VMEM((SUB,), jnp.int32),
        pltpu.VMEM((K, SUB), jnp.int32),
        pltpu.VMEM((SUB, D), jnp.float32),
        pltpu.VMEM((SUB, D), jnp.float32),
        pltpu.VMEM((SUB * K, D), jnp.float32),
        pltpu.VMEM((SUB,), jnp.int32),
        pltpu.VMEM((SUB,), jnp.float32),
        pltpu.VMEM((SUB, D), jnp.float32),
        pltpu.VMEM((L,), jnp.float32),
        pltpu.SemaphoreType.DMA,
        pltpu.SemaphoreType.DMA,
    ],
    compiler_params=pltpu.CompilerParams(needs_layout_passes=False,
                                         use_tc_tiling_on_sc=False),
)


def _flat_body(n_ref, o_ref):
    o_ref[...] = n_ref[...].reshape(K * B)


_flatten_negs = pl.pallas_call(
    _flat_body,
    out_shape=jax.ShapeDtypeStruct((K * B,), jnp.int32),
)


N_ROWS = 1000000
NTILES = 7813
NT_FULL = NTILES - 1
NPAD = NTILES * 128


def _detile_body(src_t, tail, dst, tile_v, out_v0, out_v1, rsem, wsem):
    wid = lax.axis_index("s") * NC + lax.axis_index("c")
    iota16 = lax.iota(jnp.int32, L) * D
    out_vs = (out_v0, out_v1)

    def issue_read(b, ct):
        for dt in range(2):
            pltpu.async_copy(
                src_t.at[pl.ds(dt * 8, 8), pl.ds(ct * 128, 128)],
                tile_v.at[b, dt], rsem[b])

    def wait_read(b):
        for dt in range(2):
            pltpu.make_async_copy(src_t.at[pl.ds(0, 8), pl.ds(0, 128)],
                                  tile_v.at[b, dt], rsem[b]).wait()

    def scatter(b):
        ov = out_vs[b]
        for dt in range(2):
            for d8 in range(8):
                d = dt * 8 + d8
                for g in range(8):
                    vec = tile_v[b, dt, d8, pl.ds(g * L, L)]
                    idx = iota16 + (g * L * D + d)
                    plsc.store_scatter(ov, [idx], vec)

    def issue_write(b, ct):
        pltpu.async_copy(out_vs[b], dst.at[pl.ds(ct * 2048, 2048)],
                         wsem[b])

    def wait_write(b):
        pltpu.make_async_copy(out_vs[b], dst.at[pl.ds(0, 2048)],
                              wsem[b]).wait()

    for b in range(2):
        ct0 = wid + b * NW
        @pl.when(ct0 < NT_FULL)
        def _():
            issue_read(b, ct0)

    def loop_body(i2, _):
        for b in range(2):
            ct = wid + (i2 * 2 + b) * NW

            @pl.when(ct < NT_FULL)
            def _():
                wait_read(b)

                @pl.when(ct - 2 * NW >= wid)
                def _():
                    wait_write(b)

                scatter(b)
                issue_write(b, ct)
                nct = ct + 2 * NW

                @pl.when(nct < NT_FULL)
                def _():
                    issue_read(b, nct)
        return 0

    niter = (NT_FULL // NW + 2) // 2
    lax.fori_loop(0, niter, loop_body, 0)
    for b in range(2):
        @pl.when(wid + b * NW < NT_FULL)
        def _():
            wait_write(b)

    @pl.when(wid == 0)
    def _():
        pltpu.sync_copy(tail, dst.at[pl.ds(NT_FULL * 128 * D, 64 * D)])


_detile = pl.kernel(
    _detile_body,
    out_type=jax.ShapeDtypeStruct((N_ROWS * D,), jnp.float32),
    mesh=plsc.VectorSubcoreMesh(core_axis_name="c", subcore_axis_name="s",
                                num_cores=NC, num_subcores=NS),
    scratch_types=[
        pltpu.VMEM((2, 2, 8, 128), jnp.float32),
        pltpu.VMEM((2048,), jnp.float32),
        pltpu.VMEM((2048,), jnp.float32),
        [pltpu.SemaphoreType.DMA, pltpu.SemaphoreType.DMA],
        [pltpu.SemaphoreType.DMA, pltpu.SemaphoreType.DMA],
    ],
    compiler_params=pltpu.CompilerParams(needs_layout_passes=False),
)


def _tc_body(diff_ref, sq_ref, loss_ref, reg_ref):
    z = -diff_ref[...]
    sp = jnp.maximum(z, 0.0) + jnp.log1p(jnp.exp(-jnp.abs(z)))
    loss_ref[...] = jnp.sum(sp).reshape(1, 1) / B
    reg_ref[...] = (REGS_COEF * 0.5 / B) * jnp.sum(sq_ref[...]).reshape(1, 1)


@jax.jit
def _run(user, pos, negs, user_embedding, item_embedding):
    user = user.astype(jnp.int32)
    pos = pos.astype(jnp.int32)
    negs_flat = _flatten_negs(negs.astype(jnp.int32).T)
    utail = user_embedding[NT_FULL * 128:].reshape(64 * D)
    itail = item_embedding[NT_FULL * 128:].reshape(64 * D)
    uemb_lin = _detile(user_embedding.T, utail).reshape(N_ROWS, D)
    iemb_lin = _detile(item_embedding.T, itail).reshape(N_ROWS, D)
    diff, sq = _sc_call(user, pos, negs_flat, uemb_lin, iemb_lin)
    loss, reg = pl.pallas_call(
        _tc_body,
        out_shape=(jax.ShapeDtypeStruct((1, 1), jnp.float32),
                   jax.ShapeDtypeStruct((1, 1), jnp.float32)),
    )(diff.reshape(128, 128), sq)
    return loss[0, 0], reg[0, 0]


def kernel(user, pos, negs, user_embedding, item_embedding):
    return _run(user, pos, negs, user_embedding, item_embedding)


# speedup vs baseline: 2.1681x; 1.0159x over previous
import jax
import jax.numpy as jnp
from jax import lax
from jax.experimental import pallas as pl
from jax.experimental.pallas import tpu as pltpu
from jax.experimental.pallas import tpu_sc as plsc

NC, NS, L = 2, 16, 16
NW = NC * NS
B = 16384
K = 50
D = 16
RPW = B // NW
SUB = 128
NSUB = RPW // SUB
NG = SUB // L
REGS_COEF = 1e-05


def _sc_body(user_hbm, pos_hbm, negs_hbm, uemb_hbm, iemb_hbm,
             diff_hbm, sq_hbm,
             u_idx_v, p_idx_v, n_idx_v, u_rows, p_rows, n_rows,
             bid_v, diff_v, nbest, sq_stage, sem, isem):
    wid = lax.axis_index("s") * NC + lax.axis_index("c")
    iota = lax.iota(jnp.int32, L)
    dcol = [jnp.full((L,), d, jnp.int32) for d in range(D)]

    def sub_body(s, sq_total):
        base = wid * RPW + s * SUB
        pltpu.sync_copy(user_hbm.at[pl.ds(base, SUB)], u_idx_v)
        pltpu.sync_copy(pos_hbm.at[pl.ds(base, SUB)], p_idx_v)

        def idx_body(j, _):
            pltpu.async_copy(negs_hbm.at[pl.ds(j * B + base, SUB)],
                             n_idx_v.at[j], isem)
            return 0

        lax.fori_loop(0, K, idx_body, 0)
        cu = pltpu.async_copy(uemb_hbm.at[u_idx_v], u_rows, sem)
        cp = pltpu.async_copy(iemb_hbm.at[p_idx_v], p_rows, sem)

        def idx_drain(j, _):
            pltpu.make_async_copy(negs_hbm.at[pl.ds(0, SUB)],
                                  n_idx_v.at[0], isem).wait()
            return 0

        lax.fori_loop(0, K, idx_drain, 0)

        def issue_body(j, _):
            pltpu.async_copy(iemb_hbm.at[n_idx_v.at[j]],
                             n_rows.at[pl.ds(j * SUB, SUB), :], sem)
            return 0

        lax.fori_loop(0, K, issue_body, 0)
        cu.wait()
        cp.wait()

        def drain_body(j, _):
            pltpu.make_async_copy(iemb_hbm.at[n_idx_v.at[0]],
                                  n_rows.at[pl.ds(0, SUB), :], sem).wait()
            return 0

        lax.fori_loop(0, K, drain_body, 0)

        def g_body(g, sq_acc):
            rows = g * L + iota
            ud = [plsc.load_gather(u_rows, [rows, dcol[d]]) for d in range(D)]
            ps = jnp.zeros((L,), jnp.float32)
            for d in range(D):
                pv = plsc.load_gather(p_rows, [rows, dcol[d]])
                ps = ps + ud[d] * pv
                sq_acc = sq_acc + ud[d] * ud[d] + pv * pv

            def j_body(j, c):
                best, bid = c
                frow = j * SUB + rows
                jsplat = jnp.full((L,), j, jnp.int32)
                acc = jnp.zeros((L,), jnp.float32)
                for d in range(D):
                    nv = plsc.load_gather(n_rows, [frow, dcol[d]])
                    acc = acc + ud[d] * nv
                ids = plsc.load_gather(n_idx_v, [jsplat, rows])
                m = acc > best
                return jnp.where(m, acc, best), jnp.where(m, ids, bid)

            best, bid = lax.fori_loop(
                0, K, j_body,
                (jnp.full((L,), -1e30, jnp.float32),
                 jnp.zeros((L,), jnp.int32)))
            bid_v[pl.ds(g * L, L)] = bid
            diff_v[pl.ds(g * L, L)] = ps - best
            return sq_acc

        sq_total = lax.fori_loop(0, NG, g_body, sq_total)

        cb = pltpu.async_copy(iemb_hbm.at[bid_v], nbest, sem)
        cb.wait()

        def r_body(r, sq_acc):
            v = nbest[r, :]
            return sq_acc + v * v

        sq_total = lax.fori_loop(0, SUB, r_body, sq_total)

        pltpu.sync_copy(diff_v, diff_hbm.at[pl.ds(base, SUB)])
        return sq_total

    sq_total = lax.fori_loop(0, NSUB, sub_body, jnp.zeros((L,), jnp.float32))
    sq_stage[...] = sq_total
    pltpu.sync_copy(sq_stage, sq_hbm.at[wid])


_sc_call = pl.kernel(
    _sc_body,
    out_type=(jax.ShapeDtypeStruct((B,), jnp.float32),
              jax.ShapeDtypeStruct((NW, L), jnp.float32)),
    mesh=plsc.VectorSubcoreMesh(core_axis_name="c", subcore_axis_name="s",
                                num_cores=NC, num_subcores=NS),
    scratch_types=[
        pltpu.VMEM((SUB,), jnp.int32),
        pltpu.VMEM((SUB,), jnp.int32),
        pltpu.VMEM((K, SUB), jnp.int32),
        pltpu.VMEM((SUB, D), jnp.float32),
        pltpu.VMEM((SUB, D), jnp.float32),
        pltpu.VMEM((SUB * K, D), jnp.float32),
        pltpu.VMEM((SUB,), jnp.int32),
        pltpu.VMEM((SUB,), jnp.float32),
        pltpu.VMEM((SUB, D), jnp.float32),
        pltpu.VMEM((L,), jnp.float32),
        pltpu.SemaphoreType.DMA,
        pltpu.SemaphoreType.DMA,
    ],
    compiler_params=pltpu.CompilerParams(needs_layout_passes=False,
                                         use_tc_tiling_on_sc=False),
)


def _flat_body(n_ref, o_ref):
    o_ref[...] = n_ref[...].reshape(K * B)


_flatten_negs = pl.pallas_call(
    _flat_body,
    out_shape=jax.ShapeDtypeStruct((K * B,), jnp.int32),
)


N_ROWS = 1000000
NTILES = 7813
NT_FULL = NTILES - 1
NPAD = NTILES * 128


def _detile_body(src_t, tail, dst, tile_v, out_v0, out_v1, out_v2, out_v3,
                 rsem, wsem):
    wid = lax.axis_index("s") * NC + lax.axis_index("c")
    iota16 = lax.iota(jnp.int32, L) * D
    out_vs = (out_v0, out_v1, out_v2, out_v3)
    NBUF = 4

    def issue_read(b, ct):
        for dt in range(2):
            pltpu.async_copy(
                src_t.at[pl.ds(dt * 8, 8), pl.ds(ct * 128, 128)],
                tile_v.at[b, dt], rsem[b])

    def wait_read(b):
        for dt in range(2):
            pltpu.make_async_copy(src_t.at[pl.ds(0, 8), pl.ds(0, 128)],
                                  tile_v.at[b, dt], rsem[b]).wait()

    def scatter(b):
        ov = out_vs[b]
        for dt in range(2):
            for d8 in range(8):
                d = dt * 8 + d8
                vecs = [tile_v[b, dt, d8, pl.ds(g * L, L)] for g in range(8)]
                for g in range(8):
                    idx = iota16 + (g * L * D + d)
                    plsc.store_scatter(ov, [idx], vecs[g])

    def issue_write(b, ct):
        pltpu.async_copy(out_vs[b], dst.at[pl.ds(ct * 2048, 2048)],
                         wsem[b])

    def wait_write(b):
        pltpu.make_async_copy(out_vs[b], dst.at[pl.ds(0, 2048)],
                              wsem[b]).wait()

    for b in range(NBUF):
        ct0 = wid + b * NW
        @pl.when(ct0 < NT_FULL)
        def _():
            issue_read(b, ct0)

    def loop_body(it, _):
        for b in range(NBUF):
            ct = wid + (it * NBUF + b) * NW

            @pl.when(ct < NT_FULL)
            def _():
                wait_read(b)

                @pl.when(ct - NBUF * NW >= wid)
                def _():
                    wait_write(b)

                scatter(b)
                issue_write(b, ct)
                nct = ct + NBUF * NW

                @pl.when(nct < NT_FULL)
                def _():
                    issue_read(b, nct)
        return 0

    niter = (NT_FULL // NW + NBUF) // NBUF + 1
    lax.fori_loop(0, niter, loop_body, 0)
    for b in range(NBUF):
        @pl.when(wid + b * NW < NT_FULL)
        def _():
            wait_write(b)

    @pl.when(wid == 0)
    def _():
        pltpu.sync_copy(tail, dst.at[pl.ds(NT_FULL * 128 * D, 64 * D)])


_detile = pl.kernel(
    _detile_body,
    out_type=jax.ShapeDtypeStruct((N_ROWS * D,), jnp.float32),
    mesh=plsc.VectorSubcoreMesh(core_axis_name="c", subcore_axis_name="s",
                                num_cores=NC, num_subcores=NS),
    scratch_types=[
        pltpu.VMEM((4, 2, 8, 128), jnp.float32),
        pltpu.VMEM((2048,), jnp.float32),
        pltpu.VMEM((2048,), jnp.float32),
        pltpu.VMEM((2048,), jnp.float32),
        pltpu.VMEM((2048,), jnp.float32),
        [pltpu.SemaphoreType.DMA] * 4,
        [pltpu.SemaphoreType.DMA] * 4,
    ],
    compiler_params=pltpu.CompilerParams(needs_layout_passes=False),
)


def _tc_body(diff_ref, sq_ref, loss_ref, reg_ref):
    z = -diff_ref[...]
    sp = jnp.maximum(z, 0.0) + jnp.log1p(jnp.exp(-jnp.abs(z)))
    loss_ref[...] = jnp.sum(sp).reshape(1, 1) / B
    reg_ref[...] = (REGS_COEF * 0.5 / B) * jnp.sum(sq_ref[...]).reshape(1, 1)


@jax.jit
def _run(user, pos, negs, user_embedding, item_embedding):
    user = user.astype(jnp.int32)
    pos = pos.astype(jnp.int32)
    negs_flat = _flatten_negs(negs.astype(jnp.int32).T)
    utail = user_embedding[NT_FULL * 128:].reshape(64 * D)
    itail = item_embedding[NT_FULL * 128:].reshape(64 * D)
    uemb_lin = _detile(user_embedding.T, utail).reshape(N_ROWS, D)
    iemb_lin = _detile(item_embedding.T, itail).reshape(N_ROWS, D)
    diff, sq = _sc_call(user, pos, negs_flat, uemb_lin, iemb_lin)
    loss, reg = pl.pallas_call(
        _tc_body,
        out_shape=(jax.ShapeDtypeStruct((1, 1), jnp.float32),
                   jax.ShapeDtypeStruct((1, 1), jnp.float32)),
    )(diff.reshape(128, 128), sq)
    return loss[0, 0], reg[0, 0]


def kernel(user, pos, negs, user_embedding, item_embedding):
    return _run(user, pos, negs, user_embedding, item_embedding)
